# Initial kernel scaffold; baseline (speedup 1.0000x reference)
import jax
import jax.numpy as jnp
from jax.experimental import pallas as pl


def kernel(hidden_states, gate_w, w1, w2, w3):
    raise NotImplementedError("write your pallas kernel here")


# speedup vs baseline: 2.3802x; 2.3802x over previous
import functools

import jax
import jax.numpy as jnp
from jax import lax
from jax.experimental import pallas as pl
from jax.experimental.pallas import tpu as pltpu
from jax.experimental.pallas import tpu_sc as plsc

NUM_EXPERTS = 8
TOP_K = 2
DIM = 1024
FFN = 2048
T = 2048
TT = TOP_K * T

BLK = 256
PADT = TT + NUM_EXPERTS * BLK
NB = PADT // BLK
FT = 2
FFN_T = FFN // FT

NW = 32
CH_D = TT // NW
CH_C = T // NW


def _router_body(x_ref, gw_ref, sc_ref, ex_ref):
    x = x_ref[...]
    gw = gw_ref[...]
    l = lax.dot_general(x, gw, (((1,), (1,)), ((), ())),
                        preferred_element_type=jnp.float32)
    idx = lax.broadcasted_iota(jnp.int32, l.shape, 1)
    m1 = jnp.max(l, axis=1, keepdims=True)
    a1 = jnp.min(jnp.where(l >= m1, idx, NUM_EXPERTS), axis=1)
    neg = jnp.where(idx == a1[:, None], -jnp.inf, l)
    m2 = jnp.max(neg, axis=1, keepdims=True)
    a2 = jnp.min(jnp.where(neg >= m2, idx, NUM_EXPERTS), axis=1)
    den = jnp.sum(jnp.exp(l - m1), axis=1, keepdims=True)
    s1 = 1.0 / den
    s2 = jnp.exp(m2 - m1) / den
    sc_ref[...] = jnp.concatenate([s1, s2], axis=1)
    ex_ref[...] = jnp.concatenate([a1[:, None], a2[:, None]], axis=1)


def _router(x, gate_w):
    blk = 256
    return pl.pallas_call(
        _router_body,
        grid=(T // blk,),
        in_specs=[
            pl.BlockSpec((blk, DIM), lambda b: (b, 0)),
            pl.BlockSpec((NUM_EXPERTS, DIM), lambda b: (0, 0)),
        ],
        out_specs=[
            pl.BlockSpec((blk, TOP_K), lambda b: (b, 0)),
            pl.BlockSpec((blk, TOP_K), lambda b: (b, 0)),
        ],
        out_shape=[
            jax.ShapeDtypeStruct((T, TOP_K), jnp.float32),
            jax.ShapeDtypeStruct((T, TOP_K), jnp.int32),
        ],
    )(x, gate_w)


def _dispatch_body(x_hbm, tok_hbm, pos_hbm, disp_hbm, tokv, posv, rows, sem):
    w = lax.axis_index("s") * 2 + lax.axis_index("c")
    pltpu.sync_copy(tok_hbm.at[w], tokv)
    pltpu.sync_copy(pos_hbm.at[w], posv)
    for j in range(2):
        pltpu.async_copy(x_hbm.at[tokv.at[j]], rows, sem).wait()
        pltpu.async_copy(rows, disp_hbm.at[posv.at[j]], sem).wait()


def _dispatch_sc(x, tok, pos):
    h = CH_D // 2
    k = functools.partial(
        pl.kernel,
        mesh=plsc.VectorSubcoreMesh(core_axis_name="c", subcore_axis_name="s"),
        out_type=jax.ShapeDtypeStruct((PADT, DIM), jnp.float32),
        scratch_types=[
            pltpu.VMEM((2, h), jnp.int32),
            pltpu.VMEM((2, h), jnp.int32),
            pltpu.VMEM((h, DIM), jnp.float32),
            pltpu.SemaphoreType.DMA,
        ],
    )(_dispatch_body)
    return k(x, tok.reshape(NW, 2, h), pos.reshape(NW, 2, h))


def _swiglu_body(ble_ref, nv_ref, x_ref, w1_ref, w3_ref, w2_ref, out_ref):
    b = pl.program_id(0)
    ft = pl.program_id(1)

    @pl.when(b < nv_ref[0])
    def _():
        x = x_ref[...]
        h1 = jnp.dot(x, w1_ref[0], preferred_element_type=jnp.float32)
        h3 = jnp.dot(x, w3_ref[0], preferred_element_type=jnp.float32)
        h = h1 * (1.0 / (1.0 + jnp.exp(-h1))) * h3
        y = jnp.dot(h, w2_ref[0], preferred_element_type=jnp.float32)

        @pl.when(ft == 0)
        def _():
            out_ref[...] = y

        @pl.when(ft != 0)
        def _():
            out_ref[...] += y


def _swiglu(disp, w1, w3, w2, ble, nvalid):
    grid_spec = pltpu.PrefetchScalarGridSpec(
        num_scalar_prefetch=2,
        grid=(NB, FT),
        in_specs=[
            pl.BlockSpec((BLK, DIM), lambda b, ft, ble, nv: (b, 0)),
            pl.BlockSpec((1, DIM, FFN_T), lambda b, ft, ble, nv: (ble[b], 0, ft)),
            pl.BlockSpec((1, DIM, FFN_T), lambda b, ft, ble, nv: (ble[b], 0, ft)),
            pl.BlockSpec((1, FFN_T, DIM), lambda b, ft, ble, nv: (ble[b], ft, 0)),
        ],
        out_specs=pl.BlockSpec((BLK, DIM), lambda b, ft, ble, nv: (b, 0)),
    )
    return pl.pallas_call(
        _swiglu_body,
        grid_spec=grid_spec,
        out_shape=jax.ShapeDtypeStruct((PADT, DIM), jnp.float32),
    )(ble, nvalid, disp, w1, w3, w2)


def _combine_body(eo_hbm, pos_hbm, gath_hbm, posv, buf, sem):
    w = lax.axis_index("s") * 2 + lax.axis_index("c")
    pltpu.sync_copy(pos_hbm.at[w], posv)
    for k in range(TOP_K):
        pltpu.async_copy(eo_hbm.at[posv.at[k]], buf, sem).wait()
        pltpu.sync_copy(buf, gath_hbm.at[k, pl.ds(w * CH_C, CH_C)])


def _combine_sc(eo, pos_cmb):
    k = functools.partial(
        pl.kernel,
        mesh=plsc.VectorSubcoreMesh(core_axis_name="c", subcore_axis_name="s"),
        out_type=jax.ShapeDtypeStruct((TOP_K, T, DIM), jnp.float32),
        scratch_types=[
            pltpu.VMEM((TOP_K, CH_C), jnp.int32),
            pltpu.VMEM((CH_C, DIM), jnp.float32),
            pltpu.SemaphoreType.DMA,
        ],
    )(_combine_body)
    return k(eo, pos_cmb)


def _wsum_body(g_ref, s_ref, out_ref):
    g = g_ref[...]
    s = s_ref[...]
    out_ref[...] = g[0] * s[:, 0:1] + g[1] * s[:, 1:2]


def _wsum(gath, scores):
    blk = 256
    return pl.pallas_call(
        _wsum_body,
        grid=(T // blk,),
        in_specs=[
            pl.BlockSpec((TOP_K, blk, DIM), lambda b: (0, b, 0)),
            pl.BlockSpec((blk, TOP_K), lambda b: (b, 0)),
        ],
        out_specs=pl.BlockSpec((blk, DIM), lambda b: (b, 0)),
        out_shape=jax.ShapeDtypeStruct((T, DIM), jnp.float32),
    )(gath, scores)


def _bookkeeping(experts):
    flat_e = experts.T.reshape(-1)
    oh = (flat_e[:, None] == jnp.arange(NUM_EXPERTS)[None, :]).astype(jnp.int32)
    csum = jnp.cumsum(oh, axis=0)
    counts = csum[-1]
    rank = jnp.take_along_axis(csum - oh, flat_e[:, None], axis=1)[:, 0]
    pc = ((counts + BLK - 1) // BLK) * BLK
    bounds = jnp.cumsum(pc)
    off = bounds - pc
    pos = (off[flat_e] + rank).astype(jnp.int32)
    tok = (jnp.arange(TT, dtype=jnp.int32) % T)
    nvalid = (bounds[-1] // BLK).astype(jnp.int32).reshape(1)
    ble = jnp.minimum(
        jnp.searchsorted(bounds, jnp.arange(NB) * BLK, side="right"),
        NUM_EXPERTS - 1,
    ).astype(jnp.int32)
    return pos, tok, ble, nvalid


def kernel(hidden_states, gate_w, w1, w2, w3):
    orig_shape = hidden_states.shape
    x = hidden_states.reshape(-1, DIM)

    scores, experts = _router(x, gate_w)
    pos, tok, ble, nvalid = _bookkeeping(experts)

    disp = _dispatch_sc(x, tok, pos)
    eo = _swiglu(disp, w1, w3, w2, ble, nvalid)

    pos_cmb = pos.reshape(TOP_K, NW, CH_C).transpose(1, 0, 2)
    gath = _combine_sc(eo, pos_cmb)
    out = _wsum(gath, scores)
    return out.reshape(orig_shape)
